# baseline (speedup 1.0000x reference)
import functools

import jax
import jax.numpy as jnp
from jax import lax
from jax.experimental import pallas as pl
from jax.experimental.pallas import tpu as pltpu
from jax.experimental.pallas import tpu_sc as plsc

N_NODES = 10000
E_EDGES = 320000
D_FEAT = 128

NC = 2
NS = 16
NW = NC * NS

CHUNK = 96
EPW = E_EDGES // NW
NCHUNK = 108
EPW_PAD = NCHUNK * CHUNK
ACC_ROWS = 10112
RPT = ACC_ROWS // NS


_MESH = plsc.VectorSubcoreMesh(
    core_axis_name="c", subcore_axis_name="s", num_cores=NC, num_subcores=NS)
_PARAMS = pltpu.CompilerParams(use_tc_tiling_on_sc=False)


def _zero_init_slice(sp_ref, buf, s):
    kfull, rem = RPT // CHUNK, RPT % CHUNK
    for k in range(kfull):
        pltpu.sync_copy(buf, sp_ref.at[pl.ds(s * RPT + k * CHUNK, CHUNK)])
    pltpu.sync_copy(buf.at[pl.ds(0, rem)],
                    sp_ref.at[pl.ds(s * RPT + kfull * CHUNK, rem)])


def _make_seg_sum():
    scratch = [
        pltpu.VMEM_SHARED((ACC_ROWS, D_FEAT), jnp.float32),
        pltpu.VMEM((NCHUNK // 2, CHUNK), jnp.int32),
        pltpu.VMEM((NCHUNK // 2, CHUNK), jnp.int32),
        pltpu.VMEM((CHUNK, D_FEAT), jnp.float32),
        pltpu.VMEM((CHUNK, D_FEAT), jnp.float32),
        pltpu.VMEM((CHUNK, D_FEAT), jnp.float32),
        pltpu.SemaphoreType.DMA,
        pltpu.SemaphoreType.DMA,
        pltpu.SemaphoreType.DMA,
    ]

    def body(table_hbm, src_hbm, dst_hbm, zeros_hbm, part_hbm,
             acc_sp, src_v, dst_v, rows0, rows1, rows2, sem0, sem1, sem2):
        rows = (rows0, rows1, rows2)
        sems = (sem0, sem1, sem2)
        c = lax.axis_index("c")
        s = lax.axis_index("s")
        wid = c * NS + s

        pltpu.sync_copy(zeros_hbm, rows0)
        _zero_init_slice(acc_sp, rows0, s)
        plsc.subcore_barrier()

        NB = 3
        HALF = NCHUNK // 2
        for phase in range(2):
            pltpu.sync_copy(src_hbm.at[wid, pl.ds(phase * HALF, HALF)], src_v)
            pltpu.sync_copy(dst_hbm.at[wid, pl.ds(phase * HALF, HALF)], dst_v)
            for b in range(NB):
                pltpu.async_copy(table_hbm.at[src_v.at[b]], rows[b], sems[b])

            @pl.loop(0, HALF, step=NB)
            def _(j):
                for b in range(NB):
                    pltpu.make_async_copy(
                        table_hbm.at[src_v.at[j + b]], rows[b], sems[b]).wait()
                    pltpu.sync_copy(rows[b], acc_sp.at[dst_v.at[j + b]],
                                    add=True)

                    @pl.when(j + b + NB < HALF)
                    def _():
                        pltpu.async_copy(
                            table_hbm.at[src_v.at[j + b + NB]], rows[b],
                            sems[b])

        plsc.subcore_barrier()
        sl = pl.ds(s * RPT, RPT)
        pltpu.sync_copy(acc_sp.at[sl], part_hbm.at[c, sl])

    return pl.kernel(
        body,
        out_type=jax.ShapeDtypeStruct((NC, ACC_ROWS, D_FEAT), jnp.float32),
        mesh=_MESH, scratch_types=scratch, compiler_params=_PARAMS)


def _make_count():
    scratch = [
        pltpu.VMEM_SHARED((ACC_ROWS, 16), jnp.float32),
        pltpu.VMEM((NCHUNK, CHUNK), jnp.int32),
        pltpu.VMEM((CHUNK, 16), jnp.float32),
    ]

    def body(dst_hbm, zcnt_hbm, ones_hbm, cnt_hbm, cnt_sp, dst_v, ones_v):
        c = lax.axis_index("c")
        s = lax.axis_index("s")
        wid = c * NS + s

        pltpu.sync_copy(zcnt_hbm, ones_v)
        _zero_init_slice(cnt_sp, ones_v, s)
        pltpu.sync_copy(ones_hbm, ones_v)
        pltpu.sync_copy(dst_hbm.at[wid], dst_v)
        plsc.subcore_barrier()

        @pl.loop(0, NCHUNK)
        def _(j):
            pltpu.sync_copy(ones_v, cnt_sp.at[dst_v.at[j]], add=True)

        plsc.subcore_barrier()
        sl = pl.ds(s * RPT, RPT)
        pltpu.sync_copy(cnt_sp.at[sl], cnt_hbm.at[c, sl])

    return pl.kernel(
        body,
        out_type=jax.ShapeDtypeStruct((NC, ACC_ROWS, 16), jnp.float32),
        mesh=_MESH, scratch_types=scratch, compiler_params=_PARAMS)


_seg_sum = _make_seg_sum()
_count = _make_count()


def _combine1_body(a0, a1, c0, c1, x, wl, wr, b, h_out):
    cnt = jnp.clip(c0[:, 0:1] + c1[:, 0:1], 1.0, None)
    agg = (a0[...] + a1[...]) / cnt
    h = jnp.dot(agg, wl[...], preferred_element_type=jnp.float32)
    h += jnp.dot(x[...], wr[...], preferred_element_type=jnp.float32)
    h_out[...] = jnp.maximum(h + b[...], 0.0)


def _bn(h, g, b, eps=1e-5):
    mu = jnp.mean(h, axis=0, keepdims=True)
    var = jnp.mean((h - mu) * (h - mu), axis=0, keepdims=True)
    return (h - mu) * lax.rsqrt(var + eps) * g + b


def _head_body(a0, a1, c0, c1, h1, wl, wr, b,
               fc1_w, fc1_b, bn1_g, bn1_b, fc2_w, fc2_b, bn2_g, bn2_b,
               fc3_w, fc3_b, h2_out, out_out):
    cnt = jnp.clip(c0[:, 0:1] + c1[:, 0:1], 1.0, None)
    agg = (a0[...] + a1[...]) / cnt
    h = jnp.dot(agg, wl[...], preferred_element_type=jnp.float32)
    h += jnp.dot(h1[...], wr[...], preferred_element_type=jnp.float32)
    h2 = jnp.maximum(h + b[...], 0.0)
    h2_out[...] = h2
    f = jnp.dot(h2, fc1_w[...], preferred_element_type=jnp.float32) + fc1_b[...]
    f = jnp.maximum(_bn(f, bn1_g[...], bn1_b[...]), 0.0)
    f = jnp.dot(f, fc2_w[...], preferred_element_type=jnp.float32) + fc2_b[...]
    f = jnp.maximum(_bn(f, bn2_g[...], bn2_b[...]), 0.0)
    o = jnp.dot(f, fc3_w[...], preferred_element_type=jnp.float32) + fc3_b[...]
    out_out[...] = o


def kernel(x, edge_index, conv1_wl, conv1_wr, conv1_b, conv2_wl, conv2_wr,
           conv2_b, fc1_w, fc1_b, bn1_g, bn1_b, fc2_w, fc2_b, bn2_g, bn2_b,
           fc3_w, fc3_b):
    x = x.reshape(x.shape[0], -1)

    pad = EPW_PAD * NW - E_EDGES
    src = jnp.concatenate([edge_index[0], jnp.zeros((pad,), jnp.int32)])
    pad_dst = N_NODES + (jnp.arange(pad, dtype=jnp.int32)
                         % (ACC_ROWS - N_NODES))
    dst = jnp.concatenate([edge_index[1], pad_dst])
    src_w = src.reshape(NW, NCHUNK, CHUNK)
    dst_w = dst.reshape(NW, NCHUNK, CHUNK)

    zeros = jnp.zeros((CHUNK, D_FEAT), jnp.float32)
    zcnt = jnp.zeros((CHUNK, 16), jnp.float32)
    ones = jnp.ones((CHUNK, 16), jnp.float32)

    cnt = _count(dst_w, zcnt, ones)

    part1 = _seg_sum(x, src_w, dst_w, zeros)

    h1 = pl.pallas_call(
        _combine1_body,
        out_shape=jax.ShapeDtypeStruct((N_NODES, D_FEAT), jnp.float32),
    )(part1[0, :N_NODES], part1[1, :N_NODES],
      cnt[0, :N_NODES], cnt[1, :N_NODES],
      x, conv1_wl, conv1_wr, conv1_b.reshape(1, -1))

    part2 = _seg_sum(h1, src_w, dst_w, zeros)

    h2, out = pl.pallas_call(
        _head_body,
        out_shape=[
            jax.ShapeDtypeStruct((N_NODES, D_FEAT), jnp.float32),
            jax.ShapeDtypeStruct((N_NODES, 1), jnp.float32),
        ],
    )(part2[0, :N_NODES], part2[1, :N_NODES],
      cnt[0, :N_NODES], cnt[1, :N_NODES],
      h1, conv2_wl, conv2_wr, conv2_b.reshape(1, -1),
      fc1_w, fc1_b.reshape(1, -1), bn1_g.reshape(1, -1), bn1_b.reshape(1, -1),
      fc2_w, fc2_b.reshape(1, -1), bn2_g.reshape(1, -1), bn2_b.reshape(1, -1),
      fc3_w, fc3_b.reshape(1, -1))

    return (out[:, 0], h1, h2)


# speedup vs baseline: 2.5140x; 2.5140x over previous
import functools

import jax
import jax.numpy as jnp
from jax import lax
from jax.experimental import pallas as pl
from jax.experimental.pallas import tpu as pltpu
from jax.experimental.pallas import tpu_sc as plsc

N_NODES = 10000
E_EDGES = 320000
D_FEAT = 128

NC = 2
NS = 16
NW = NC * NS

CHUNK = 112
EPW = E_EDGES // NW
NCHUNK = 90
EPW_PAD = NCHUNK * CHUNK
ACC_ROWS = 10112
RPT = ACC_ROWS // NS


_MESH = plsc.VectorSubcoreMesh(
    core_axis_name="c", subcore_axis_name="s", num_cores=NC, num_subcores=NS)
_PARAMS = pltpu.CompilerParams(use_tc_tiling_on_sc=False)


def _zero_init_slice(sp_ref, buf, s):
    kfull, rem = RPT // CHUNK, RPT % CHUNK
    for k in range(kfull):
        pltpu.sync_copy(buf, sp_ref.at[pl.ds(s * RPT + k * CHUNK, CHUNK)])
    pltpu.sync_copy(buf.at[pl.ds(0, rem)],
                    sp_ref.at[pl.ds(s * RPT + kfull * CHUNK, rem)])


def _make_seg_sum():
    scratch = [
        pltpu.VMEM_SHARED((ACC_ROWS, D_FEAT), jnp.float32),
        pltpu.VMEM((NCHUNK, CHUNK), jnp.int32),
        pltpu.VMEM((NCHUNK, CHUNK), jnp.int32),
        pltpu.VMEM((CHUNK, D_FEAT), jnp.float32),
        pltpu.VMEM((CHUNK, D_FEAT), jnp.float32),
        pltpu.SemaphoreType.DMA,
        pltpu.SemaphoreType.DMA,
    ]

    def body(table_hbm, src_hbm, dst_hbm, zeros_hbm, part_hbm,
             acc_sp, src_v, dst_v, rows0, rows1, sem0, sem1):
        rows = (rows0, rows1)
        sems = (sem0, sem1)
        c = lax.axis_index("c")
        s = lax.axis_index("s")
        wid = c * NS + s

        pltpu.sync_copy(zeros_hbm, rows0)
        _zero_init_slice(acc_sp, rows0, s)
        pltpu.sync_copy(src_hbm.at[wid], src_v)
        pltpu.sync_copy(dst_hbm.at[wid], dst_v)
        plsc.subcore_barrier()

        for b in range(2):
            pltpu.async_copy(table_hbm.at[src_v.at[b]], rows[b], sems[b])

        @pl.loop(0, NCHUNK, step=2)
        def _(j):
            for b in range(2):
                pltpu.make_async_copy(
                    table_hbm.at[src_v.at[j + b]], rows[b], sems[b]).wait()
                pltpu.sync_copy(rows[b], acc_sp.at[dst_v.at[j + b]], add=True)

                @pl.when(j + b + 2 < NCHUNK)
                def _():
                    pltpu.async_copy(
                        table_hbm.at[src_v.at[j + b + 2]], rows[b], sems[b])

        plsc.subcore_barrier()
        sl = pl.ds(s * RPT, RPT)
        pltpu.sync_copy(acc_sp.at[sl], part_hbm.at[c, sl])

    return pl.kernel(
        body,
        out_type=jax.ShapeDtypeStruct((NC, ACC_ROWS, D_FEAT), jnp.float32),
        mesh=_MESH, scratch_types=scratch, compiler_params=_PARAMS)


def _make_count():
    scratch = [
        pltpu.VMEM_SHARED((ACC_ROWS, 16), jnp.float32),
        pltpu.VMEM((NCHUNK, CHUNK), jnp.int32),
        pltpu.VMEM((CHUNK, 16), jnp.float32),
    ]

    def body(dst_hbm, zcnt_hbm, ones_hbm, cnt_hbm, cnt_sp, dst_v, ones_v):
        c = lax.axis_index("c")
        s = lax.axis_index("s")
        wid = c * NS + s

        pltpu.sync_copy(zcnt_hbm, ones_v)
        _zero_init_slice(cnt_sp, ones_v, s)
        pltpu.sync_copy(ones_hbm, ones_v)
        pltpu.sync_copy(dst_hbm.at[wid], dst_v)
        plsc.subcore_barrier()

        @pl.loop(0, NCHUNK)
        def _(j):
            pltpu.sync_copy(ones_v, cnt_sp.at[dst_v.at[j]], add=True)

        plsc.subcore_barrier()
        sl = pl.ds(s * RPT, RPT)
        pltpu.sync_copy(cnt_sp.at[sl], cnt_hbm.at[c, sl])

    return pl.kernel(
        body,
        out_type=jax.ShapeDtypeStruct((NC, ACC_ROWS, 16), jnp.float32),
        mesh=_MESH, scratch_types=scratch, compiler_params=_PARAMS)


_seg_sum = _make_seg_sum()
_count = _make_count()


def _combine1_body(a0, a1, c0, c1, x, wl, wr, b, h_out):
    cnt = jnp.clip(c0[:, 0:1] + c1[:, 0:1], 1.0, None)
    agg = (a0[...] + a1[...]) / cnt
    h = jnp.dot(agg, wl[...], preferred_element_type=jnp.float32)
    h += jnp.dot(x[...], wr[...], preferred_element_type=jnp.float32)
    h_out[...] = jnp.maximum(h + b[...], 0.0)


def _bn(h, g, b, eps=1e-5):
    mu = jnp.mean(h, axis=0, keepdims=True)
    var = jnp.mean((h - mu) * (h - mu), axis=0, keepdims=True)
    return (h - mu) * lax.rsqrt(var + eps) * g + b


def _head_body(a0, a1, c0, c1, h1, wl, wr, b,
               fc1_w, fc1_b, bn1_g, bn1_b, fc2_w, fc2_b, bn2_g, bn2_b,
               fc3_w, fc3_b, h2_out, out_out):
    cnt = jnp.clip(c0[:, 0:1] + c1[:, 0:1], 1.0, None)
    agg = (a0[...] + a1[...]) / cnt
    h = jnp.dot(agg, wl[...], preferred_element_type=jnp.float32)
    h += jnp.dot(h1[...], wr[...], preferred_element_type=jnp.float32)
    h2 = jnp.maximum(h + b[...], 0.0)
    h2_out[...] = h2
    f = jnp.dot(h2, fc1_w[...], preferred_element_type=jnp.float32) + fc1_b[...]
    f = jnp.maximum(_bn(f, bn1_g[...], bn1_b[...]), 0.0)
    f = jnp.dot(f, fc2_w[...], preferred_element_type=jnp.float32) + fc2_b[...]
    f = jnp.maximum(_bn(f, bn2_g[...], bn2_b[...]), 0.0)
    o = jnp.dot(f, fc3_w[...], preferred_element_type=jnp.float32) + fc3_b[...]
    out_out[...] = o


def kernel(x, edge_index, conv1_wl, conv1_wr, conv1_b, conv2_wl, conv2_wr,
           conv2_b, fc1_w, fc1_b, bn1_g, bn1_b, fc2_w, fc2_b, bn2_g, bn2_b,
           fc3_w, fc3_b):
    x = x.reshape(x.shape[0], -1)

    pad = EPW_PAD * NW - E_EDGES
    src = jnp.concatenate([edge_index[0], jnp.zeros((pad,), jnp.int32)])
    pad_dst = N_NODES + (jnp.arange(pad, dtype=jnp.int32)
                         % (ACC_ROWS - N_NODES))
    dst = jnp.concatenate([edge_index[1], pad_dst])
    src_w = src.reshape(NW, NCHUNK, CHUNK)
    dst_w = dst.reshape(NW, NCHUNK, CHUNK)

    zeros = jnp.zeros((CHUNK, D_FEAT), jnp.float32)
    zcnt = jnp.zeros((CHUNK, 16), jnp.float32)
    ones = jnp.ones((CHUNK, 16), jnp.float32)

    cnt = _count(dst_w, zcnt, ones)

    part1 = _seg_sum(x, src_w, dst_w, zeros)

    h1 = pl.pallas_call(
        _combine1_body,
        out_shape=jax.ShapeDtypeStruct((N_NODES, D_FEAT), jnp.float32),
    )(part1[0, :N_NODES], part1[1, :N_NODES],
      cnt[0, :N_NODES], cnt[1, :N_NODES],
      x, conv1_wl, conv1_wr, conv1_b.reshape(1, -1))

    part2 = _seg_sum(h1, src_w, dst_w, zeros)

    h2, out = pl.pallas_call(
        _head_body,
        out_shape=[
            jax.ShapeDtypeStruct((N_NODES, D_FEAT), jnp.float32),
            jax.ShapeDtypeStruct((N_NODES, 1), jnp.float32),
        ],
    )(part2[0, :N_NODES], part2[1, :N_NODES],
      cnt[0, :N_NODES], cnt[1, :N_NODES],
      h1, conv2_wl, conv2_wr, conv2_b.reshape(1, -1),
      fc1_w, fc1_b.reshape(1, -1), bn1_g.reshape(1, -1), bn1_b.reshape(1, -1),
      fc2_w, fc2_b.reshape(1, -1), bn2_g.reshape(1, -1), bn2_b.reshape(1, -1),
      fc3_w, fc3_b.reshape(1, -1))

    return (out[:, 0], h1, h2)


# speedup vs baseline: 2.7844x; 1.1075x over previous
import functools

import jax
import jax.numpy as jnp
from jax import lax
from jax.experimental import pallas as pl
from jax.experimental.pallas import tpu as pltpu
from jax.experimental.pallas import tpu_sc as plsc

N_NODES = 10000
E_EDGES = 320000
D_FEAT = 128

NC = 2
NS = 16
NW = NC * NS

CHUNK = 112
EPW = E_EDGES // NW
NCHUNK = 90
EPW_PAD = NCHUNK * CHUNK
ACC_ROWS = 10112
RPT = ACC_ROWS // NS

BIG_C = 0
PHASE = 60
NPH_BIG = 2
NPH_SML = 1
E_BIG = NS * PHASE * NPH_BIG * CHUNK
E_SML = NS * PHASE * NPH_SML * CHUNK


_MESH = plsc.VectorSubcoreMesh(
    core_axis_name="c", subcore_axis_name="s", num_cores=NC, num_subcores=NS)
_PARAMS = pltpu.CompilerParams(use_tc_tiling_on_sc=False)


def _zero_init_slice(sp_ref, buf, s):
    kfull, rem = RPT // CHUNK, RPT % CHUNK
    for k in range(kfull):
        pltpu.sync_copy(buf, sp_ref.at[pl.ds(s * RPT + k * CHUNK, CHUNK)])
    pltpu.sync_copy(buf.at[pl.ds(0, rem)],
                    sp_ref.at[pl.ds(s * RPT + kfull * CHUNK, rem)])


def _make_seg_sum():
    scratch = [
        pltpu.VMEM_SHARED((ACC_ROWS, D_FEAT), jnp.float32),
        pltpu.VMEM((PHASE, CHUNK), jnp.int32),
        pltpu.VMEM((PHASE, CHUNK), jnp.int32),
        pltpu.VMEM((CHUNK, D_FEAT), jnp.float32),
        pltpu.VMEM((CHUNK, D_FEAT), jnp.float32),
        pltpu.SemaphoreType.DMA,
        pltpu.SemaphoreType.DMA,
    ]

    def body(table_hbm, srcb_hbm, dstb_hbm, srcs_hbm, dsts_hbm, zeros_hbm,
             part_hbm, acc_sp, src_v, dst_v, rows0, rows1, sem0, sem1):
        rows = (rows0, rows1)
        sems = (sem0, sem1)
        c = lax.axis_index("c")
        s = lax.axis_index("s")

        pltpu.sync_copy(zeros_hbm, rows0)
        _zero_init_slice(acc_sp, rows0, s)
        plsc.subcore_barrier()

        def run_phase(src_hbm_slice, dst_hbm_slice):
            pltpu.sync_copy(src_hbm_slice, src_v)
            pltpu.sync_copy(dst_hbm_slice, dst_v)
            for b in range(2):
                pltpu.async_copy(table_hbm.at[src_v.at[b]], rows[b], sems[b])

            @pl.loop(0, PHASE, step=2)
            def _(j):
                for b in range(2):
                    pltpu.make_async_copy(
                        table_hbm.at[src_v.at[j + b]], rows[b], sems[b]).wait()
                    pltpu.sync_copy(rows[b], acc_sp.at[dst_v.at[j + b]],
                                    add=True)

                    @pl.when(j + b + 2 < PHASE)
                    def _():
                        pltpu.async_copy(
                            table_hbm.at[src_v.at[j + b + 2]], rows[b],
                            sems[b])

        @pl.when(c == BIG_C)
        def _():
            for p in range(NPH_BIG):
                run_phase(srcb_hbm.at[s, pl.ds(p * PHASE, PHASE)],
                          dstb_hbm.at[s, pl.ds(p * PHASE, PHASE)])

        @pl.when(c != BIG_C)
        def _():
            for p in range(NPH_SML):
                run_phase(srcs_hbm.at[s, pl.ds(p * PHASE, PHASE)],
                          dsts_hbm.at[s, pl.ds(p * PHASE, PHASE)])

        plsc.subcore_barrier()
        sl = pl.ds(s * RPT, RPT)
        pltpu.sync_copy(acc_sp.at[sl], part_hbm.at[c, sl])

    return pl.kernel(
        body,
        out_type=jax.ShapeDtypeStruct((NC, ACC_ROWS, D_FEAT), jnp.float32),
        mesh=_MESH, scratch_types=scratch, compiler_params=_PARAMS)


def _make_count():
    scratch = [
        pltpu.VMEM_SHARED((ACC_ROWS, 16), jnp.float32),
        pltpu.VMEM((NCHUNK, CHUNK), jnp.int32),
        pltpu.VMEM((CHUNK, 16), jnp.float32),
    ]

    def body(dst_hbm, zcnt_hbm, ones_hbm, cnt_hbm, cnt_sp, dst_v, ones_v):
        c = lax.axis_index("c")
        s = lax.axis_index("s")
        wid = c * NS + s

        pltpu.sync_copy(zcnt_hbm, ones_v)
        _zero_init_slice(cnt_sp, ones_v, s)
        pltpu.sync_copy(ones_hbm, ones_v)
        pltpu.sync_copy(dst_hbm.at[wid], dst_v)
        plsc.subcore_barrier()

        @pl.loop(0, NCHUNK)
        def _(j):
            pltpu.sync_copy(ones_v, cnt_sp.at[dst_v.at[j]], add=True)

        plsc.subcore_barrier()
        sl = pl.ds(s * RPT, RPT)
        pltpu.sync_copy(cnt_sp.at[sl], cnt_hbm.at[c, sl])

    return pl.kernel(
        body,
        out_type=jax.ShapeDtypeStruct((NC, ACC_ROWS, 16), jnp.float32),
        mesh=_MESH, scratch_types=scratch, compiler_params=_PARAMS)


_seg_sum = _make_seg_sum()
_count = _make_count()


def _combine1_body(a0, a1, c0, c1, x, wl, wr, b, h_out):
    cnt = jnp.clip(c0[:, 0:1] + c1[:, 0:1], 1.0, None)
    agg = (a0[...] + a1[...]) / cnt
    h = jnp.dot(agg, wl[...], preferred_element_type=jnp.float32)
    h += jnp.dot(x[...], wr[...], preferred_element_type=jnp.float32)
    h_out[...] = jnp.maximum(h + b[...], 0.0)


def _bn(h, g, b, eps=1e-5):
    mu = jnp.mean(h, axis=0, keepdims=True)
    var = jnp.mean((h - mu) * (h - mu), axis=0, keepdims=True)
    return (h - mu) * lax.rsqrt(var + eps) * g + b


def _head_body(a0, a1, c0, c1, h1, wl, wr, b,
               fc1_w, fc1_b, bn1_g, bn1_b, fc2_w, fc2_b, bn2_g, bn2_b,
               fc3_w, fc3_b, h2_out, out_out):
    cnt = jnp.clip(c0[:, 0:1] + c1[:, 0:1], 1.0, None)
    agg = (a0[...] + a1[...]) / cnt
    h = jnp.dot(agg, wl[...], preferred_element_type=jnp.float32)
    h += jnp.dot(h1[...], wr[...], preferred_element_type=jnp.float32)
    h2 = jnp.maximum(h + b[...], 0.0)
    h2_out[...] = h2
    f = jnp.dot(h2, fc1_w[...], preferred_element_type=jnp.float32) + fc1_b[...]
    f = jnp.maximum(_bn(f, bn1_g[...], bn1_b[...]), 0.0)
    f = jnp.dot(f, fc2_w[...], preferred_element_type=jnp.float32) + fc2_b[...]
    f = jnp.maximum(_bn(f, bn2_g[...], bn2_b[...]), 0.0)
    o = jnp.dot(f, fc3_w[...], preferred_element_type=jnp.float32) + fc3_b[...]
    out_out[...] = o


def kernel(x, edge_index, conv1_wl, conv1_wr, conv1_b, conv2_wl, conv2_wr,
           conv2_b, fc1_w, fc1_b, bn1_g, bn1_b, fc2_w, fc2_b, bn2_g, bn2_b,
           fc3_w, fc3_b):
    x = x.reshape(x.shape[0], -1)

    def padded(n_total):
        pad = n_total - E_EDGES
        s_ = jnp.concatenate([edge_index[0], jnp.zeros((pad,), jnp.int32)])
        pad_dst = N_NODES + (jnp.arange(pad, dtype=jnp.int32)
                             % (ACC_ROWS - N_NODES))
        d_ = jnp.concatenate([edge_index[1], pad_dst])
        return s_, d_

    src_c, dst_c = padded(EPW_PAD * NW)
    dst_w = dst_c.reshape(NW, NCHUNK, CHUNK)
    src_a, dst_a = padded(E_BIG + E_SML)
    src_b = src_a[:E_BIG].reshape(NS, NPH_BIG * PHASE, CHUNK)
    dst_b = dst_a[:E_BIG].reshape(NS, NPH_BIG * PHASE, CHUNK)
    src_s = src_a[E_BIG:].reshape(NS, NPH_SML * PHASE, CHUNK)
    dst_s = dst_a[E_BIG:].reshape(NS, NPH_SML * PHASE, CHUNK)

    zeros = jnp.zeros((CHUNK, D_FEAT), jnp.float32)
    zcnt = jnp.zeros((CHUNK, 16), jnp.float32)
    ones = jnp.ones((CHUNK, 16), jnp.float32)

    cnt = _count(dst_w, zcnt, ones)

    part1 = _seg_sum(x, src_b, dst_b, src_s, dst_s, zeros)

    h1 = pl.pallas_call(
        _combine1_body,
        out_shape=jax.ShapeDtypeStruct((N_NODES, D_FEAT), jnp.float32),
    )(part1[0, :N_NODES], part1[1, :N_NODES],
      cnt[0, :N_NODES], cnt[1, :N_NODES],
      x, conv1_wl, conv1_wr, conv1_b.reshape(1, -1))

    part2 = _seg_sum(h1, src_b, dst_b, src_s, dst_s, zeros)

    h2, out = pl.pallas_call(
        _head_body,
        out_shape=[
            jax.ShapeDtypeStruct((N_NODES, D_FEAT), jnp.float32),
            jax.ShapeDtypeStruct((N_NODES, 1), jnp.float32),
        ],
    )(part2[0, :N_NODES], part2[1, :N_NODES],
      cnt[0, :N_NODES], cnt[1, :N_NODES],
      h1, conv2_wl, conv2_wr, conv2_b.reshape(1, -1),
      fc1_w, fc1_b.reshape(1, -1), bn1_g.reshape(1, -1), bn1_b.reshape(1, -1),
      fc2_w, fc2_b.reshape(1, -1), bn2_g.reshape(1, -1), bn2_b.reshape(1, -1),
      fc3_w, fc3_b.reshape(1, -1))

    return (out[:, 0], h1, h2)
